# baseline (speedup 1.0000x reference)
import functools

import jax
import jax.numpy as jnp
import numpy as np
from jax import lax
from jax.experimental import pallas as pl
from jax.experimental.pallas import tpu as pltpu
from jax.experimental.pallas import tpu_sc as plsc

STEP = 200
DIM = 16
BATCH = 1024
VOCAB = 1000000

NC = 2
NS = 16
NW = NC * NS

NB = 64
NP = 100
VREG = 16


def _pe_table() -> np.ndarray:
    pos = np.arange(STEP)[:, None]
    with np.errstate(divide="ignore", invalid="ignore", over="ignore"):
        pe = pos / (np.power(1000, 2 * np.arange(DIM, dtype=np.int64))[None, :] / DIM)
        pe[:, 0::2] = np.sin(pe[:, 0::2])
        pe[:, 0::1] = np.cos(pe[:, 0::1])
    return pe.astype(np.float32)


_PE_NP = _pe_table()
_PE_DUP = np.ascontiguousarray(
    np.broadcast_to(_PE_NP.T[:, :, None], (DIM, STEP, VREG))
)
_DUMMY = np.zeros((DIM, STEP, NB), np.float32)


def _sc_body(xt_hbm, w_hbm, pe_hbm, zd_hbm, out_hbm, idx_v, pe_v, rows_v, sem, wsem):
    wid = lax.axis_index("s") * NC + lax.axis_index("c")
    j = wid // 4
    h = (wid // 2) % 2
    q = wid % 2
    b0 = j * 128 + h * NB
    p0 = q * NP

    pltpu.sync_copy(xt_hbm.at[pl.ds(p0, NP), pl.ds(b0, NB)], idx_v)

    def _issue(p, carry):
        for d in range(DIM):
            pltpu.async_copy(
                w_hbm.at[d].at[idx_v.at[p]],
                rows_v.at[d, p],
                sem,
            )
        return carry

    lax.fori_loop(0, NP, _issue, 0)
    pltpu.make_async_copy(zd_hbm.at[:, pl.ds(0, NP)], rows_v, sem).wait()

    for half in range(2):
        ph = NP // 2 * half
        pltpu.sync_copy(pe_hbm.at[:, pl.ds(p0 + ph, NP // 2)], pe_v)

        def _add(p, carry):
            for d in range(DIM):
                v = pe_v[d, p]
                for k in range(NB // VREG):
                    rows_v[d, ph + p, pl.ds(k * VREG, VREG)] = (
                        rows_v[d, ph + p, pl.ds(k * VREG, VREG)] + v
                    )
            return carry

        lax.fori_loop(0, NP // 2, _add, 0)

    for d in range(DIM):
        pltpu.async_copy(
            rows_v.at[d],
            out_hbm.at[pl.ds(p0, NP), d // 8, j, d % 8, pl.ds(h * NB, NB)],
            wsem,
        )
    pltpu.make_async_copy(zd_hbm.at[:, pl.ds(0, NP)], rows_v, wsem).wait()


@functools.partial(jax.jit, static_argnames=())
def _sc_gather_pe(xt, w, pe_dup, zd):
    mesh = plsc.VectorSubcoreMesh(core_axis_name="c", subcore_axis_name="s")
    call = pl.kernel(
        _sc_body,
        mesh=mesh,
        out_type=jax.ShapeDtypeStruct((STEP, 2, 8, 8, 128), jnp.float32),
        scratch_types=[
            pltpu.VMEM((NP, NB), jnp.int32),
            pltpu.VMEM((DIM, NP // 2, VREG), jnp.float32),
            pltpu.VMEM((DIM, NP, NB), jnp.float32),
            pltpu.SemaphoreType.DMA,
            pltpu.SemaphoreType.DMA,
        ],
        compiler_params=pltpu.CompilerParams(use_tc_tiling_on_sc=False),
    )
    return call(xt, w, pe_dup, zd)


def kernel(x, table):
    xt = jnp.swapaxes(x.astype(jnp.int32), 0, 1)
    w = jnp.swapaxes(table, 0, 1)
    pe_dup = jnp.asarray(_PE_DUP)
    zd = jnp.asarray(_DUMMY)
    out5 = _sc_gather_pe(xt, w, pe_dup, zd)
    return out5.transpose(2, 4, 0, 1, 3).reshape(BATCH, STEP, DIM)


# speedup vs baseline: 2.4214x; 2.4214x over previous
import functools

import jax
import jax.numpy as jnp
import numpy as np
from jax import lax
from jax.experimental import pallas as pl
from jax.experimental.pallas import tpu as pltpu
from jax.experimental.pallas import tpu_sc as plsc

STEP = 200
DIM = 16
BATCH = 1024

NC = 2
NS = 16
NW = NC * NS

SEQ_PER_W = BATCH // NW
CHUNKS = ((0, 128), (128, 72))


def _pe_table() -> np.ndarray:
    pos = np.arange(STEP)[:, None]
    with np.errstate(divide="ignore", invalid="ignore", over="ignore"):
        pe = pos / (np.power(1000, 2 * np.arange(DIM, dtype=np.int64))[None, :] / DIM)
        pe[:, 0::2] = np.sin(pe[:, 0::2])
        pe[:, 0::1] = np.cos(pe[:, 0::1])
    return pe.astype(np.float32)


_PE_NP = _pe_table()


def _sc_body(x_hbm, table_hbm, pe_hbm, out_hbm, idx_v, pe_v, rows_v, sem, psem):
    wid = lax.axis_index("s") * NC + lax.axis_index("c")
    base = wid * SEQ_PER_W

    pltpu.sync_copy(x_hbm.at[pl.ds(base, SEQ_PER_W)], idx_v)
    pltpu.sync_copy(pe_hbm, pe_v)

    def _prefill(p, carry):
        v = pe_v[p]

        def _store(s, c):
            rows_v[s, p] = v
            return c

        return lax.fori_loop(0, SEQ_PER_W, _store, carry)

    lax.fori_loop(0, STEP, _prefill, 0)

    def _issue(s, carry):
        for off, ln in CHUNKS:
            pltpu.async_copy(
                table_hbm.at[idx_v.at[s, pl.ds(off, ln)]],
                rows_v.at[s, pl.ds(off, ln)],
                sem,
                add=True,
            )
        return carry

    lax.fori_loop(0, SEQ_PER_W, _issue, 0)
    pltpu.make_async_copy(out_hbm.at[pl.ds(base, SEQ_PER_W)], rows_v, sem).wait()

    pltpu.sync_copy(rows_v, out_hbm.at[pl.ds(base, SEQ_PER_W)])


@functools.partial(jax.jit, static_argnames=())
def _sc_gather_pe(x, table, pe):
    mesh = plsc.VectorSubcoreMesh(core_axis_name="c", subcore_axis_name="s")
    call = pl.kernel(
        _sc_body,
        mesh=mesh,
        out_type=jax.ShapeDtypeStruct((BATCH, STEP, DIM), jnp.float32),
        scratch_types=[
            pltpu.VMEM((SEQ_PER_W, STEP), jnp.int32),
            pltpu.VMEM((STEP, DIM), jnp.float32),
            pltpu.VMEM((SEQ_PER_W, STEP, DIM), jnp.float32),
            pltpu.SemaphoreType.DMA,
            pltpu.SemaphoreType.DMA,
        ],
        compiler_params=pltpu.CompilerParams(use_tc_tiling_on_sc=False),
    )
    return call(x, table, pe)


def kernel(x, table):
    pe = jnp.asarray(_PE_NP)
    return _sc_gather_pe(x.astype(jnp.int32), table, pe)
